# initial kernel scaffold (unmeasured)
import jax
import jax.numpy as jnp
from jax import lax
from jax.experimental import pallas as pl
from jax.experimental.pallas import tpu as pltpu

B = 32
H = 16
D = 128
BS = 32
NBT = 256
NPAGE = 256
T = NPAGE * BS
PACK = 256
NEG = -1e30


def _partial_body(q_ref, k_ref, v_ref, bt_ref, lens_ref, part_ref, cnt_tok):
    h = pl.program_id(0)

    @pl.when(h == 0)
    def _():
        my_x = lax.axis_index("x")
        pid = my_x * NPAGE + lax.broadcasted_iota(jnp.int32, (B, NBT, NPAGE), 2)
        bt3 = bt_ref[...][:, :, None]
        jidx = lax.broadcasted_iota(jnp.int32, (B, NBT, NPAGE), 1)
        valid = jidx < lens_ref[...][:, :, None]
        cnt_pages = jnp.sum(
            jnp.where((bt3 == pid) & valid, 1.0, 0.0), axis=1
        )
        tp = lax.broadcasted_iota(jnp.int32, (NPAGE, T), 1) // BS
        pp = lax.broadcasted_iota(jnp.int32, (NPAGE, T), 0)
        expand = jnp.where(tp == pp, 1.0, 0.0).astype(jnp.bfloat16)
        cnt_tok[...] = jnp.dot(
            cnt_pages.astype(jnp.bfloat16), expand,
            preferred_element_type=jnp.float32,
        )

    q = q_ref[:, 0, 0, :].astype(jnp.bfloat16)
    k = k_ref[...].reshape(T, D).astype(jnp.bfloat16)
    v = v_ref[...].reshape(T, D).astype(jnp.bfloat16)
    s = lax.dot_general(
        q, k, (((1,), (1,)), ((), ())), preferred_element_type=jnp.float32
    ) * (D ** -0.5)
    cnt = cnt_tok[...]
    s = jnp.where(cnt > 0.0, s, NEG)
    m = jnp.max(s, axis=1, keepdims=True)
    p = jnp.exp(s - m) * cnt
    l = jnp.sum(p, axis=1, keepdims=True)
    o = lax.dot_general(
        p.astype(jnp.bfloat16), v, (((1,), (0,)), ((), ())),
        preferred_element_type=jnp.float32,
    )
    part_ref[0, :, 0:D] = o
    part_ref[0, :, D:D + 1] = m
    part_ref[0, :, D + 1:D + 2] = l
    part_ref[0, :, D + 2:PACK] = jnp.zeros((B, PACK - D - 2), jnp.float32)


def _combine_body(part_ref, out_ref, rx_ref, send_sem, recv_sem):
    my_x = lax.axis_index("x")
    my_y = lax.axis_index("y")
    my_z = lax.axis_index("z")
    peer = (1 - my_x, my_y, my_z)

    bar = pltpu.get_barrier_semaphore()
    pl.semaphore_signal(
        bar, inc=1, device_id=peer, device_id_type=pl.DeviceIdType.MESH
    )
    pl.semaphore_wait(bar, 1)

    rdma = pltpu.make_async_remote_copy(
        src_ref=part_ref,
        dst_ref=rx_ref,
        send_sem=send_sem,
        recv_sem=recv_sem,
        device_id=peer,
        device_id_type=pl.DeviceIdType.MESH,
    )
    rdma.start()
    rdma.wait()

    o_l = part_ref[:, :, 0:D]
    m_l = part_ref[:, :, D:D + 1]
    l_l = part_ref[:, :, D + 1:D + 2]
    o_r = rx_ref[:, :, 0:D]
    m_r = rx_ref[:, :, D:D + 1]
    l_r = rx_ref[:, :, D + 1:D + 2]

    m_c = jnp.maximum(m_l, m_r)
    a_l = jnp.exp(m_l - m_c)
    a_r = jnp.exp(m_r - m_c)
    l_c = l_l * a_l + l_r * a_r
    o = (o_l * a_l + o_r * a_r) / l_c
    for hh in range(H):
        out_ref[:, 0, hh, :] = o[hh]


def kernel(Q, K, V, bt, lens):
    lens2 = lens.reshape(B, 1)

    part = pl.pallas_call(
        _partial_body,
        grid=(H,),
        in_specs=[
            pl.BlockSpec((B, 1, 1, D), lambda h: (0, 0, h, 0)),
            pl.BlockSpec((NPAGE, BS, 1, D), lambda h: (0, 0, h, 0)),
            pl.BlockSpec((NPAGE, BS, 1, D), lambda h: (0, 0, h, 0)),
            pl.BlockSpec((B, NBT), lambda h: (0, 0)),
            pl.BlockSpec((B, 1), lambda h: (0, 0)),
        ],
        out_specs=pl.BlockSpec((1, B, PACK), lambda h: (h, 0, 0)),
        out_shape=jax.ShapeDtypeStruct((H, B, PACK), jnp.float32),
        scratch_shapes=[pltpu.VMEM((B, T), jnp.float32)],
    )(Q, K, V, bt, lens2)

    return pl.pallas_call(
        _combine_body,
        out_shape=jax.ShapeDtypeStruct((B, 1, H, D), jnp.float32),
        in_specs=[pl.BlockSpec(memory_space=pltpu.VMEM)],
        out_specs=pl.BlockSpec(memory_space=pltpu.VMEM),
        scratch_shapes=[
            pltpu.VMEM((H, B, PACK), jnp.float32),
            pltpu.SemaphoreType.DMA,
            pltpu.SemaphoreType.DMA,
        ],
        compiler_params=pltpu.CompilerParams(collective_id=0),
    )(part)


# baseline (device time: 197569 ns/iter reference)
import jax
import jax.numpy as jnp
from jax import lax
from jax.experimental import pallas as pl
from jax.experimental.pallas import tpu as pltpu

B = 32
H = 16
D = 128
BS = 32
NBT = 256
NPAGE = 256
T = NPAGE * BS
PC = 16
NC = NPAGE // PC
TC = PC * BS
PACK = 256
NEG = -1e30


def _count_body(bt_ref, lens_ref, cnt_ref):
    my_x = lax.axis_index("x")
    pid = my_x * NPAGE + lax.broadcasted_iota(jnp.int32, (B, NBT, NPAGE), 2)
    bt3 = bt_ref[...][:, :, None]
    jidx = lax.broadcasted_iota(jnp.int32, (B, NBT, NPAGE), 1)
    valid = jidx < lens_ref[...][:, :, None]
    cnt_pages = jnp.sum(
        jnp.where((bt3 == pid) & valid, 1.0, 0.0), axis=1
    )
    tp = lax.broadcasted_iota(jnp.int32, (NPAGE, T), 1) // BS
    pp = lax.broadcasted_iota(jnp.int32, (NPAGE, T), 0)
    expand = jnp.where(tp == pp, 1.0, 0.0).astype(jnp.bfloat16)
    cnt_ref[...] = jnp.dot(
        cnt_pages.astype(jnp.bfloat16), expand,
        preferred_element_type=jnp.float32,
    )


def _partial_body(q_ref, k_ref, v_ref, cnt_ref, part_ref, acc_ref):
    c = pl.program_id(0)

    @pl.when(c == 0)
    def _():
        acc_ref[...] = jnp.zeros((H, B, PACK), jnp.float32)
        acc_ref[:, :, D:D + 1] = jnp.full((H, B, 1), NEG, jnp.float32)

    cnt = cnt_ref[...]
    for h in range(H):
        q = q_ref[:, 0, h, :].astype(jnp.bfloat16)
        k = k_ref[:, :, h, :].reshape(TC, D).astype(jnp.bfloat16)
        v = v_ref[:, :, h, :].reshape(TC, D).astype(jnp.bfloat16)
        s = lax.dot_general(
            q, k, (((1,), (1,)), ((), ())),
            preferred_element_type=jnp.float32,
        ) * (D ** -0.5)
        s = jnp.where(cnt > 0.0, s, NEG)

        m_old = acc_ref[h, :, D:D + 1]
        l_old = acc_ref[h, :, D + 1:D + 2]
        o_old = acc_ref[h, :, 0:D]
        m_new = jnp.maximum(m_old, jnp.max(s, axis=1, keepdims=True))
        p = jnp.exp(s - m_new) * cnt
        scale = jnp.exp(m_old - m_new)
        l_new = l_old * scale + jnp.sum(p, axis=1, keepdims=True)
        o_new = o_old * scale + lax.dot_general(
            p.astype(jnp.bfloat16), v, (((1,), (0,)), ((), ())),
            preferred_element_type=jnp.float32,
        )
        acc_ref[h, :, 0:D] = o_new
        acc_ref[h, :, D:D + 1] = m_new
        acc_ref[h, :, D + 1:D + 2] = l_new

    @pl.when(c == NC - 1)
    def _():
        part_ref[...] = acc_ref[...]


def _combine_body(part_ref, out_ref, rx_ref, send_sem, recv_sem):
    my_x = lax.axis_index("x")
    my_y = lax.axis_index("y")
    my_z = lax.axis_index("z")
    peer = (1 - my_x, my_y, my_z)

    bar = pltpu.get_barrier_semaphore()
    pl.semaphore_signal(
        bar, inc=1, device_id=peer, device_id_type=pl.DeviceIdType.MESH
    )
    pl.semaphore_wait(bar, 1)

    rdma = pltpu.make_async_remote_copy(
        src_ref=part_ref,
        dst_ref=rx_ref,
        send_sem=send_sem,
        recv_sem=recv_sem,
        device_id=peer,
        device_id_type=pl.DeviceIdType.MESH,
    )
    rdma.start()
    rdma.wait()

    o_l = part_ref[:, :, 0:D]
    m_l = part_ref[:, :, D:D + 1]
    l_l = part_ref[:, :, D + 1:D + 2]
    o_r = rx_ref[:, :, 0:D]
    m_r = rx_ref[:, :, D:D + 1]
    l_r = rx_ref[:, :, D + 1:D + 2]

    m_c = jnp.maximum(m_l, m_r)
    a_l = jnp.exp(m_l - m_c)
    a_r = jnp.exp(m_r - m_c)
    l_c = l_l * a_l + l_r * a_r
    o = (o_l * a_l + o_r * a_r) / l_c
    for hh in range(H):
        out_ref[:, 0, hh, :] = o[hh]


def kernel(Q, K, V, bt, lens):
    lens2 = lens.reshape(B, 1)

    cnt_tok = pl.pallas_call(
        _count_body,
        in_specs=[
            pl.BlockSpec(memory_space=pltpu.VMEM),
            pl.BlockSpec(memory_space=pltpu.VMEM),
        ],
        out_specs=pl.BlockSpec(memory_space=pltpu.VMEM),
        out_shape=jax.ShapeDtypeStruct((B, T), jnp.float32),
    )(bt, lens2)

    part = pl.pallas_call(
        _partial_body,
        grid=(NC,),
        in_specs=[
            pl.BlockSpec((B, 1, H, D), lambda c: (0, 0, 0, 0)),
            pl.BlockSpec((PC, BS, H, D), lambda c: (c, 0, 0, 0)),
            pl.BlockSpec((PC, BS, H, D), lambda c: (c, 0, 0, 0)),
            pl.BlockSpec((B, TC), lambda c: (0, c)),
        ],
        out_specs=pl.BlockSpec((H, B, PACK), lambda c: (0, 0, 0)),
        out_shape=jax.ShapeDtypeStruct((H, B, PACK), jnp.float32),
        scratch_shapes=[pltpu.VMEM((H, B, PACK), jnp.float32)],
    )(Q, K, V, cnt_tok)

    return pl.pallas_call(
        _combine_body,
        out_shape=jax.ShapeDtypeStruct((B, 1, H, D), jnp.float32),
        in_specs=[pl.BlockSpec(memory_space=pltpu.VMEM)],
        out_specs=pl.BlockSpec(memory_space=pltpu.VMEM),
        scratch_shapes=[
            pltpu.VMEM((H, B, PACK), jnp.float32),
            pltpu.SemaphoreType.DMA,
            pltpu.SemaphoreType.DMA,
        ],
        compiler_params=pltpu.CompilerParams(collective_id=0),
    )(part)


# device time: 176611 ns/iter; 1.1187x vs baseline; 1.1187x over previous
import jax
import jax.numpy as jnp
from jax import lax
from jax.experimental import pallas as pl
from jax.experimental.pallas import tpu as pltpu

B = 32
H = 16
D = 128
BS = 32
NBT = 256
NPAGE = 256
T = NPAGE * BS
PC = 32
NC = NPAGE // PC
TC = PC * BS
PACK = 256
NEG = -1e30


def _count_body(bt_ref, lens_ref, cnt_ref):
    my_x = lax.axis_index("x")
    pid = my_x * NPAGE + lax.broadcasted_iota(jnp.int32, (B, NBT, NPAGE), 2)
    bt3 = bt_ref[...][:, :, None]
    jidx = lax.broadcasted_iota(jnp.int32, (B, NBT, NPAGE), 1)
    valid = jidx < lens_ref[...][:, :, None]
    cnt_pages = jnp.sum(
        jnp.where((bt3 == pid) & valid, 1.0, 0.0), axis=1
    )
    tp = lax.broadcasted_iota(jnp.int32, (NPAGE, T), 1) // BS
    pp = lax.broadcasted_iota(jnp.int32, (NPAGE, T), 0)
    expand = jnp.where(tp == pp, 1.0, 0.0).astype(jnp.bfloat16)
    cnt_ref[...] = jnp.dot(
        cnt_pages.astype(jnp.bfloat16), expand,
        preferred_element_type=jnp.float32,
    )


def _partial_body(q_ref, k_ref, v_ref, cnt_ref, part_ref, acc_ref):
    c = pl.program_id(0)

    @pl.when(c == 0)
    def _():
        acc_ref[...] = jnp.zeros((H, B, PACK), jnp.float32)
        acc_ref[:, :, D:D + 1] = jnp.full((H, B, 1), NEG, jnp.float32)

    cnt = cnt_ref[...]
    qb = (q_ref[...] * (D ** -0.5)).astype(jnp.bfloat16)
    kb = k_ref[...].astype(jnp.bfloat16)
    vb = v_ref[...].astype(jnp.bfloat16)
    for h in range(H):
        q = qb[:, h * D:(h + 1) * D]
        k = kb[:, h * D:(h + 1) * D]
        v = vb[:, h * D:(h + 1) * D]
        s = lax.dot_general(
            q, k, (((1,), (1,)), ((), ())),
            preferred_element_type=jnp.float32,
        )
        s = jnp.where(cnt > 0.0, s, NEG)

        m_old = acc_ref[h, :, D:D + 1]
        l_old = acc_ref[h, :, D + 1:D + 2]
        o_old = acc_ref[h, :, 0:D]
        m_new = jnp.maximum(m_old, jnp.max(s, axis=1, keepdims=True))
        p = jnp.exp(s - m_new) * cnt
        scale = jnp.exp(m_old - m_new)
        l_new = l_old * scale + jnp.sum(p, axis=1, keepdims=True)
        o_new = o_old * scale + lax.dot_general(
            p.astype(jnp.bfloat16), v, (((1,), (0,)), ((), ())),
            preferred_element_type=jnp.float32,
        )
        acc_ref[h, :, 0:D] = o_new
        acc_ref[h, :, D:D + 1] = m_new
        acc_ref[h, :, D + 1:D + 2] = l_new

    @pl.when(c == NC - 1)
    def _():
        part_ref[...] = acc_ref[...]


def _combine_body(part_ref, out_ref, rx_ref, send_sem, recv_sem):
    my_x = lax.axis_index("x")
    my_y = lax.axis_index("y")
    my_z = lax.axis_index("z")
    peer = (1 - my_x, my_y, my_z)

    bar = pltpu.get_barrier_semaphore()
    pl.semaphore_signal(
        bar, inc=1, device_id=peer, device_id_type=pl.DeviceIdType.MESH
    )
    pl.semaphore_wait(bar, 1)

    rdma = pltpu.make_async_remote_copy(
        src_ref=part_ref,
        dst_ref=rx_ref,
        send_sem=send_sem,
        recv_sem=recv_sem,
        device_id=peer,
        device_id_type=pl.DeviceIdType.MESH,
    )
    rdma.start()
    rdma.wait()

    o_l = part_ref[:, :, 0:D]
    m_l = part_ref[:, :, D:D + 1]
    l_l = part_ref[:, :, D + 1:D + 2]
    o_r = rx_ref[:, :, 0:D]
    m_r = rx_ref[:, :, D:D + 1]
    l_r = rx_ref[:, :, D + 1:D + 2]

    m_c = jnp.maximum(m_l, m_r)
    a_l = jnp.exp(m_l - m_c)
    a_r = jnp.exp(m_r - m_c)
    l_c = l_l * a_l + l_r * a_r
    o = (o_l * a_l + o_r * a_r) / l_c
    for hh in range(H):
        out_ref[:, 0, hh, :] = o[hh]


def kernel(Q, K, V, bt, lens):
    lens2 = lens.reshape(B, 1)
    Q2 = Q.reshape(B, H * D)
    K2 = K.reshape(T, H * D)
    V2 = V.reshape(T, H * D)

    cnt_tok = pl.pallas_call(
        _count_body,
        in_specs=[
            pl.BlockSpec(memory_space=pltpu.VMEM),
            pl.BlockSpec(memory_space=pltpu.VMEM),
        ],
        out_specs=pl.BlockSpec(memory_space=pltpu.VMEM),
        out_shape=jax.ShapeDtypeStruct((B, T), jnp.float32),
    )(bt, lens2)

    part = pl.pallas_call(
        _partial_body,
        grid=(NC,),
        in_specs=[
            pl.BlockSpec((B, H * D), lambda c: (0, 0)),
            pl.BlockSpec((TC, H * D), lambda c: (c, 0)),
            pl.BlockSpec((TC, H * D), lambda c: (c, 0)),
            pl.BlockSpec((B, TC), lambda c: (0, c)),
        ],
        out_specs=pl.BlockSpec((H, B, PACK), lambda c: (0, 0, 0)),
        out_shape=jax.ShapeDtypeStruct((H, B, PACK), jnp.float32),
        scratch_shapes=[pltpu.VMEM((H, B, PACK), jnp.float32)],
        compiler_params=pltpu.CompilerParams(
            vmem_limit_bytes=60 * 1024 * 1024
        ),
    )(Q2, K2, V2, cnt_tok)

    return pl.pallas_call(
        _combine_body,
        out_shape=jax.ShapeDtypeStruct((B, 1, H, D), jnp.float32),
        in_specs=[pl.BlockSpec(memory_space=pltpu.VMEM)],
        out_specs=pl.BlockSpec(memory_space=pltpu.VMEM),
        scratch_shapes=[
            pltpu.VMEM((H, B, PACK), jnp.float32),
            pltpu.SemaphoreType.DMA,
            pltpu.SemaphoreType.DMA,
        ],
        compiler_params=pltpu.CompilerParams(collective_id=0),
    )(part)


# device time: 87891 ns/iter; 2.2479x vs baseline; 2.0094x over previous
import jax
import jax.numpy as jnp
from jax import lax
from jax.experimental import pallas as pl
from jax.experimental.pallas import tpu as pltpu

B = 32
H = 16
D = 128
BS = 32
NBT = 256
NPAGE = 256
T = NPAGE * BS
PC = 32
NC = NPAGE // PC
TC = PC * BS
PACK = 256
NEG = -1e30


def _count_body(bt_ref, lens_ref, cnt_ref):
    my_x = lax.axis_index("x")
    pid = my_x * NPAGE + lax.broadcasted_iota(jnp.int32, (B, NBT, NPAGE), 2)
    bt3 = bt_ref[...][:, :, None]
    jidx = lax.broadcasted_iota(jnp.int32, (B, NBT, NPAGE), 1)
    valid = jidx < lens_ref[...][:, :, None]
    cnt_pages = jnp.sum(
        jnp.where((bt3 == pid) & valid, 1.0, 0.0), axis=1
    )
    tp = lax.broadcasted_iota(jnp.int32, (NPAGE, T), 1) // BS
    pp = lax.broadcasted_iota(jnp.int32, (NPAGE, T), 0)
    expand = jnp.where(tp == pp, 1.0, 0.0).astype(jnp.bfloat16)
    cnt_ref[...] = jnp.dot(
        cnt_pages.astype(jnp.bfloat16), expand,
        preferred_element_type=jnp.float32,
    )


def _partial_body(q_ref, k_ref, v_ref, cnt_ref, part_ref, acc_ref):
    c = pl.program_id(0)

    @pl.when(c == 0)
    def _():
        acc_ref[...] = jnp.zeros((H, B, PACK), jnp.float32)
        acc_ref[:, :, D:D + 1] = jnp.full((H, B, 1), NEG, jnp.float32)

    cnt = cnt_ref[...]
    qt = jnp.swapaxes(
        (q_ref[...] * (D ** -0.5)).astype(jnp.bfloat16), 0, 1
    )
    kt = jnp.swapaxes(k_ref[...].astype(jnp.bfloat16), 0, 1)
    vt = jnp.swapaxes(v_ref[...].astype(jnp.bfloat16), 0, 1)
    for h in range(H):
        q = qt[h]
        k = kt[h]
        v = vt[h]
        s = lax.dot_general(
            q, k, (((1,), (1,)), ((), ())),
            preferred_element_type=jnp.float32,
        )
        s = jnp.where(cnt > 0.0, s, NEG)

        m_old = acc_ref[h, :, D:D + 1]
        l_old = acc_ref[h, :, D + 1:D + 2]
        o_old = acc_ref[h, :, 0:D]
        m_new = jnp.maximum(m_old, jnp.max(s, axis=1, keepdims=True))
        p = jnp.exp(s - m_new) * cnt
        scale = jnp.exp(m_old - m_new)
        l_new = l_old * scale + jnp.sum(p, axis=1, keepdims=True)
        o_new = o_old * scale + lax.dot_general(
            p.astype(jnp.bfloat16), v, (((1,), (0,)), ((), ())),
            preferred_element_type=jnp.float32,
        )
        acc_ref[h, :, 0:D] = o_new
        acc_ref[h, :, D:D + 1] = m_new
        acc_ref[h, :, D + 1:D + 2] = l_new

    @pl.when(c == NC - 1)
    def _():
        part_ref[...] = acc_ref[...]


def _combine_body(part_ref, out_ref, rx_ref, send_sem, recv_sem):
    my_x = lax.axis_index("x")
    my_y = lax.axis_index("y")
    my_z = lax.axis_index("z")
    peer = (1 - my_x, my_y, my_z)

    bar = pltpu.get_barrier_semaphore()
    pl.semaphore_signal(
        bar, inc=1, device_id=peer, device_id_type=pl.DeviceIdType.MESH
    )
    pl.semaphore_wait(bar, 1)

    rdma = pltpu.make_async_remote_copy(
        src_ref=part_ref,
        dst_ref=rx_ref,
        send_sem=send_sem,
        recv_sem=recv_sem,
        device_id=peer,
        device_id_type=pl.DeviceIdType.MESH,
    )
    rdma.start()
    rdma.wait()

    o_l = part_ref[:, :, 0:D]
    m_l = part_ref[:, :, D:D + 1]
    l_l = part_ref[:, :, D + 1:D + 2]
    o_r = rx_ref[:, :, 0:D]
    m_r = rx_ref[:, :, D:D + 1]
    l_r = rx_ref[:, :, D + 1:D + 2]

    m_c = jnp.maximum(m_l, m_r)
    a_l = jnp.exp(m_l - m_c)
    a_r = jnp.exp(m_r - m_c)
    l_c = l_l * a_l + l_r * a_r
    o = (o_l * a_l + o_r * a_r) / l_c
    for hh in range(H):
        out_ref[:, 0, hh, :] = o[hh]


def kernel(Q, K, V, bt, lens):
    lens2 = lens.reshape(B, 1)
    Q2 = Q.reshape(B, H, D)
    K2 = K.reshape(T, H, D)
    V2 = V.reshape(T, H, D)

    cnt_tok = pl.pallas_call(
        _count_body,
        in_specs=[
            pl.BlockSpec(memory_space=pltpu.VMEM),
            pl.BlockSpec(memory_space=pltpu.VMEM),
        ],
        out_specs=pl.BlockSpec(memory_space=pltpu.VMEM),
        out_shape=jax.ShapeDtypeStruct((B, T), jnp.float32),
    )(bt, lens2)

    part = pl.pallas_call(
        _partial_body,
        grid=(NC,),
        in_specs=[
            pl.BlockSpec((B, H, D), lambda c: (0, 0, 0)),
            pl.BlockSpec((TC, H, D), lambda c: (c, 0, 0)),
            pl.BlockSpec((TC, H, D), lambda c: (c, 0, 0)),
            pl.BlockSpec((B, TC), lambda c: (0, c)),
        ],
        out_specs=pl.BlockSpec((H, B, PACK), lambda c: (0, 0, 0)),
        out_shape=jax.ShapeDtypeStruct((H, B, PACK), jnp.float32),
        scratch_shapes=[pltpu.VMEM((H, B, PACK), jnp.float32)],
        compiler_params=pltpu.CompilerParams(
            vmem_limit_bytes=60 * 1024 * 1024
        ),
    )(Q2, K2, V2, cnt_tok)

    return pl.pallas_call(
        _combine_body,
        out_shape=jax.ShapeDtypeStruct((B, 1, H, D), jnp.float32),
        in_specs=[pl.BlockSpec(memory_space=pltpu.VMEM)],
        out_specs=pl.BlockSpec(memory_space=pltpu.VMEM),
        scratch_shapes=[
            pltpu.VMEM((H, B, PACK), jnp.float32),
            pltpu.SemaphoreType.DMA,
            pltpu.SemaphoreType.DMA,
        ],
        compiler_params=pltpu.CompilerParams(collective_id=0),
    )(part)
